# baseline (speedup 1.0000x reference)
import functools

import jax
import jax.numpy as jnp
from jax.experimental import pallas as pl

_TN = 512


def _chamfer_tile_kernel(x_ref, y_ref, dx_ref, dy_ref):
    i = pl.program_id(1)
    xb = x_ref[0]
    yb = y_ref[0]
    xy = jax.lax.dot_general(
        xb, yb, (((0,), (0,)), ((), ())),
        preferred_element_type=jnp.float32,
    )
    x2 = jnp.sum(xb * xb, axis=0)
    y2 = jnp.sum(yb * yb, axis=0)
    d = x2[:, None] + y2[None, :] - 2.0 * xy
    d = jnp.maximum(d, 0.0)
    dx_ref[0, 0, pl.ds(i * _TN, _TN)] = jnp.min(d, axis=1)
    col_min = jnp.min(d, axis=0)

    @pl.when(i == 0)
    def _init():
        dy_ref[0, 0, :] = col_min

    @pl.when(i > 0)
    def _acc():
        dy_ref[0, 0, :] = jnp.minimum(dy_ref[0, 0, :], col_min)


@jax.jit
def kernel(x, y):
    B, C, N = x.shape
    M = y.shape[2]
    grid = (B, N // _TN)
    dx, dy = pl.pallas_call(
        _chamfer_tile_kernel,
        grid=grid,
        in_specs=[
            pl.BlockSpec((1, C, _TN), lambda b, i: (b, 0, i)),
            pl.BlockSpec((1, C, M), lambda b, i: (b, 0, 0)),
        ],
        out_specs=[
            pl.BlockSpec((1, 1, N), lambda b, i: (b, 0, 0)),
            pl.BlockSpec((1, 1, M), lambda b, i: (b, 0, 0)),
        ],
        out_shape=[
            jax.ShapeDtypeStruct((B, 1, N), jnp.float32),
            jax.ShapeDtypeStruct((B, 1, M), jnp.float32),
        ],
    )(x, y)
    return jnp.mean(dx) + jnp.mean(dy) * 10.0


# speedup vs baseline: 1.2717x; 1.2717x over previous
import jax
import jax.numpy as jnp
from jax.experimental import pallas as pl

_TN = 512


def _chamfer_tile_kernel(x_ref, y_ref, x2_ref, y2_ref, dx_ref, dy_ref):
    i = pl.program_id(1)
    xb = x_ref[0]
    yb = y_ref[0]
    z = jax.lax.dot_general(
        xb, yb, (((0,), (0,)), ((), ())),
        preferred_element_type=jnp.float32,
    )
    x2 = x2_ref[0, 0, pl.ds(i * _TN, _TN)]
    y2 = y2_ref[0, 0, :]
    row_min = jnp.min(z + y2[None, :], axis=1)
    dx_ref[0, 0, pl.ds(i * _TN, _TN)] = row_min + x2
    col_min = jnp.min(z + x2[:, None], axis=0)

    @pl.when(i == 0)
    def _init():
        dy_ref[0, 0, :] = col_min

    @pl.when(i > 0)
    def _acc():
        dy_ref[0, 0, :] = jnp.minimum(dy_ref[0, 0, :], col_min)


@jax.jit
def kernel(x, y):
    B, C, N = x.shape
    M = y.shape[2]
    x2 = jnp.sum(x * x, axis=1)[:, None, :]
    y2 = jnp.sum(y * y, axis=1)[:, None, :]
    xbf = (-2.0 * x).astype(jnp.bfloat16)
    ybf = y.astype(jnp.bfloat16)
    grid = (B, N // _TN)
    dx, dy = pl.pallas_call(
        _chamfer_tile_kernel,
        grid=grid,
        in_specs=[
            pl.BlockSpec((1, C, _TN), lambda b, i: (b, 0, i)),
            pl.BlockSpec((1, C, M), lambda b, i: (b, 0, 0)),
            pl.BlockSpec((1, 1, N), lambda b, i: (b, 0, 0)),
            pl.BlockSpec((1, 1, M), lambda b, i: (b, 0, 0)),
        ],
        out_specs=[
            pl.BlockSpec((1, 1, N), lambda b, i: (b, 0, 0)),
            pl.BlockSpec((1, 1, M), lambda b, i: (b, 0, 0)),
        ],
        out_shape=[
            jax.ShapeDtypeStruct((B, 1, N), jnp.float32),
            jax.ShapeDtypeStruct((B, 1, M), jnp.float32),
        ],
    )(xbf, ybf, x2, y2)
    dx = jnp.maximum(dx, 0.0)
    dy = jnp.maximum(dy + y2, 0.0)
    return jnp.mean(dx) + jnp.mean(dy) * 10.0
